# Initial kernel scaffold; baseline (speedup 1.0000x reference)
import jax
import jax.numpy as jnp
from jax.experimental import pallas as pl


def kernel(x, pos, batch, params):
    raise NotImplementedError("write your pallas kernel here")


# speedup vs baseline: 9.7920x; 9.7920x over previous
import functools

import jax
import jax.numpy as jnp
from jax.experimental import pallas as pl
from jax.experimental.pallas import tpu as pltpu

B = 16
P0 = 1024
K = 16
BIG = 1e10
F32 = jnp.float32


def _dot(a, b):
    return jnp.dot(a, b, preferred_element_type=F32)


def _relu(a):
    return jnp.maximum(a, 0.0)


def _tb_kernel(x_ref, pos_ref, posT_ref,
               miW_ref, mib_ref,
               Win_ref, bin_ref, Wout_ref, bout_ref,
               Wlin_ref, Wsrc_ref, Wdst_ref,
               pW1_ref, pb1_ref, pW2_ref, pb2_ref,
               aW1_ref, ab1_ref, aW2_ref, ab2_ref,
               o_ref, *, P, C, has_mlp_in):
    x = x_ref[...]
    if has_mlp_in:
        miW = miW_ref[...]
        x = _relu(x[:, 0:1] * miW[0:1, :] + x[:, 1:2] * miW[1:2, :]
                  + x[:, 2:3] * miW[2:3, :] + mib_ref[...])
    xin = _relu(_dot(x, Win_ref[...]) + bin_ref[...])
    a_dst = _dot(xin, Wdst_ref[...])

    px = pos_ref[:, 0:1]
    py = pos_ref[:, 1:2]
    dx = px - posT_ref[0, 0:1, :]
    dy = py - posT_ref[0, 1:2, :]
    ii = jax.lax.broadcasted_iota(jnp.int32, (P, P), 0)
    jj = jax.lax.broadcasted_iota(jnp.int32, (P, P), 1)
    d = dx * dx + dy * dy + jnp.where(ii == jj, BIG, 0.0)
    jjf = jj.astype(F32)

    G = jnp.concatenate([pos_ref[...], xin], axis=1)

    pW1 = pW1_ref[...]
    pb1 = pb1_ref[...]
    pW2 = pW2_ref[...]
    pb2 = pb2_ref[...]
    aW1 = aW1_ref[...]
    ab1 = ab1_ref[...]
    aW2 = aW2_ref[...]
    ab2 = ab2_ref[...]
    Wsrc = Wsrc_ref[...]
    Wlin = Wlin_ref[...]

    def slot(g):
        pdx = px - g[:, 0:1]
        pdy = py - g[:, 1:2]
        gxin = g[:, 2:]
        h1 = _relu(pdx * pW1[0:1, :] + pdy * pW1[1:2, :] + pb1)
        delta = _relu(_dot(h1, pW2) + pb2)
        asrc = _dot(gxin, Wsrc)
        xl = _dot(gxin, Wlin)
        ah = _relu(_dot(a_dst - asrc + delta, aW1) + ab1)
        alpha = _relu(_dot(ah, aW2) + ab2)
        return alpha, xl + delta

    alpha0, msg0 = slot(G)
    amax0 = alpha0
    den0 = jnp.ones((P, C), F32)
    num0 = msg0

    def body(t, carry):
        d, amax, den, num = carry
        m = jnp.min(d, axis=1, keepdims=True)
        cand = jnp.where(d == m, jjf, float(P))
        jsel = jnp.min(cand, axis=1, keepdims=True)
        oh = (jjf == jsel).astype(F32)
        g = _dot(oh, G)
        d = jnp.where(oh != 0.0, BIG, d)
        alpha, msg = slot(g)
        namax = jnp.maximum(amax, alpha)
        scale = jnp.exp(amax - namax)
        e = jnp.exp(alpha - namax)
        den = den * scale + e
        num = jnp.maximum(num * scale, e * msg)
        return d, namax, den, num

    _, _, den, num = jax.lax.fori_loop(0, K, body, (d, amax0, den0, num0))
    out = num / (den + 1e-16)
    o_ref[...] = _relu(_dot(out, Wout_ref[...]) + bout_ref[...])


def _tb_block(x, pos_flat, posT, p, P, C, mlp_in=None):
    Cin = x.shape[1]
    if mlp_in is None:
        miW = jnp.zeros((4, C), F32)
        mib = jnp.zeros((1, C), F32)
        has_mlp_in = False
    else:
        miW, mib = mlp_in
        mib = mib.reshape(1, -1)
        has_mlp_in = True

    def b2(v):
        return v.reshape(1, -1)

    ws = [miW, mib,
          p['Win'], b2(p['bin']), p['Wout'], b2(p['bout']),
          p['Wlin'], p['Wsrc'], p['Wdst'],
          p['pW1'], b2(p['pb1']), p['pW2'], b2(p['pb2']),
          p['aW1'], b2(p['ab1']), p['aW2'], b2(p['ab2'])]
    w_specs = [pl.BlockSpec(w.shape, lambda b: (0, 0)) for w in ws]
    return pl.pallas_call(
        functools.partial(_tb_kernel, P=P, C=C, has_mlp_in=has_mlp_in),
        grid=(B,),
        in_specs=[pl.BlockSpec((P, Cin), lambda b: (b, 0)),
                  pl.BlockSpec((P, 2), lambda b: (b, 0)),
                  pl.BlockSpec((1, 2, P), lambda b: (b, 0, 0))] + w_specs,
        out_specs=pl.BlockSpec((P, C), lambda b: (b, 0)),
        out_shape=jax.ShapeDtypeStruct((B * P, C), F32),
    )(x, pos_flat, posT, *ws)


def _fps_kernel(pbx_ref, pby_ref, ox_ref, oy_ref, *, Pc, NS):
    pbx = pbx_ref[...]
    pby = pby_ref[...]
    jjf = jax.lax.broadcasted_iota(jnp.int32, (B, Pc), 1).astype(F32)
    sel = jax.lax.broadcasted_iota(jnp.int32, (B, NS), 1).astype(F32)

    lpx0 = pbx[:, 0:1]
    lpy0 = pby[:, 0:1]
    posx0 = jnp.where(sel == 0.0, lpx0, 0.0)
    posy0 = jnp.where(sel == 0.0, lpy0, 0.0)

    def body(i, carry):
        dists, lpx, lpy, posx, posy = carry
        ddx = pbx - lpx
        ddy = pby - lpy
        dcur = ddx * ddx + ddy * ddy
        dists = jnp.minimum(dists, dcur)
        mx = jnp.max(dists, axis=1, keepdims=True)
        cand = jnp.where(dists == mx, jjf, float(Pc))
        jselv = jnp.min(cand, axis=1, keepdims=True)
        oh = (jjf == jselv).astype(F32)
        lpx = jnp.sum(pbx * oh, axis=1, keepdims=True)
        lpy = jnp.sum(pby * oh, axis=1, keepdims=True)
        slot = (sel == i.astype(F32)).astype(F32)
        posx = posx + slot * lpx
        posy = posy + slot * lpy
        return dists, lpx, lpy, posx, posy

    init = (jnp.full((B, Pc), jnp.inf, F32), lpx0, lpy0, posx0, posy0)
    _, _, _, posx, posy = jax.lax.fori_loop(1, NS, body, init)
    ox_ref[...] = posx
    oy_ref[...] = posy


def _fps(pbx, pby, NS):
    Pc = pbx.shape[1]
    return pl.pallas_call(
        functools.partial(_fps_kernel, Pc=Pc, NS=NS),
        in_specs=[pl.BlockSpec((B, Pc), lambda: (0, 0)),
                  pl.BlockSpec((B, Pc), lambda: (0, 0))],
        out_specs=[pl.BlockSpec((B, NS), lambda: (0, 0)),
                   pl.BlockSpec((B, NS), lambda: (0, 0))],
        out_shape=[jax.ShapeDtypeStruct((B, NS), F32),
                   jax.ShapeDtypeStruct((B, NS), F32)],
    )(pbx, pby)


def _td_kernel(x_ref, posT_ref, q_ref, W_ref, b_ref, o_ref, *, P, NS):
    h = _dot(x_ref[...], W_ref[...]) + b_ref[...]
    m = jnp.mean(h, axis=0, keepdims=True)
    v = jnp.mean((h - m) ** 2, axis=0, keepdims=True)
    hn = _relu((h - m) / jnp.sqrt(v + 1e-5))

    qx = q_ref[:, 0:1]
    qy = q_ref[:, 1:2]
    ddx = qx - posT_ref[0, 0:1, :]
    ddy = qy - posT_ref[0, 1:2, :]
    d = ddx * ddx + ddy * ddy
    jjf = jax.lax.broadcasted_iota(jnp.int32, (NS, P), 1).astype(F32)

    Cout = hn.shape[1]

    def body(t, carry):
        d, pooled = carry
        mn = jnp.min(d, axis=1, keepdims=True)
        cand = jnp.where(d == mn, jjf, float(P))
        jsel = jnp.min(cand, axis=1, keepdims=True)
        oh = (jjf == jsel).astype(F32)
        g = _dot(oh, hn)
        pooled = jnp.maximum(pooled, g)
        d = jnp.where(oh != 0.0, BIG, d)
        return d, pooled

    init = (d, jnp.full((NS, Cout), -jnp.inf, F32))
    _, pooled = jax.lax.fori_loop(0, K + 1, body, init)
    o_ref[...] = pooled


def _td(x, posT, q_flat, Wb, P, NS):
    Cin = x.shape[1]
    Cout = Wb['W'].shape[1]
    b2 = Wb['b'].reshape(1, -1)
    return pl.pallas_call(
        functools.partial(_td_kernel, P=P, NS=NS),
        grid=(B,),
        in_specs=[pl.BlockSpec((P, Cin), lambda b: (b, 0)),
                  pl.BlockSpec((1, 2, P), lambda b: (b, 0, 0)),
                  pl.BlockSpec((NS, 2), lambda b: (b, 0)),
                  pl.BlockSpec(Wb['W'].shape, lambda b: (0, 0)),
                  pl.BlockSpec(b2.shape, lambda b: (0, 0))],
        out_specs=pl.BlockSpec((NS, Cout), lambda b: (b, 0)),
        out_shape=jax.ShapeDtypeStruct((B * NS, Cout), F32),
    )(x, posT, q_flat, Wb['W'], b2)


def _head_kernel(x_ref, W1_ref, b1_ref, W2_ref, b2_ref, o_ref, *, NP):
    x = x_ref[...]
    row = jax.lax.broadcasted_iota(jnp.int32, (B, B * NP), 0)
    col = jax.lax.broadcasted_iota(jnp.int32, (B, B * NP), 1)
    Msel = jnp.where(col // NP == row, 1.0, 0.0)
    xg = _dot(Msel, x) / float(NP)
    h = _relu(_dot(xg, W1_ref[...]) + b1_ref[...])
    o_ref[...] = _dot(h, W2_ref[...]) + b2_ref[...]


def _head(x, params, NP):
    C = x.shape[1]
    W1 = params['mo_W1']
    b1 = params['mo_b1'].reshape(1, -1)
    W2 = params['mo_W2']
    b2 = params['mo_b2'].reshape(1, -1)
    OUT = W2.shape[1]
    return pl.pallas_call(
        functools.partial(_head_kernel, NP=NP),
        in_specs=[pl.BlockSpec((B * NP, C), lambda: (0, 0)),
                  pl.BlockSpec(W1.shape, lambda: (0, 0)),
                  pl.BlockSpec(b1.shape, lambda: (0, 0)),
                  pl.BlockSpec(W2.shape, lambda: (0, 0)),
                  pl.BlockSpec(b2.shape, lambda: (0, 0))],
        out_specs=pl.BlockSpec((B, OUT), lambda: (0, 0)),
        out_shape=jax.ShapeDtypeStruct((B, OUT), F32),
    )(x, W1, b1, W2, b2)


def kernel(x, pos, batch, params):
    pb = pos.reshape(B, P0, 2)
    posT = pb.transpose(0, 2, 1)
    pbx = pb[:, :, 0]
    pby = pb[:, :, 1]

    x1 = _tb_block(x, pos, posT, params['tb0'], P0, 64,
                   mlp_in=(params['mi_W'], params['mi_b']))

    NS1 = 256
    sx1, sy1 = _fps(pbx, pby, NS1)
    pos1 = jnp.stack([sx1, sy1], axis=-1)
    pos1_flat = pos1.reshape(B * NS1, 2)
    pos1T = jnp.stack([sx1, sy1], axis=1)
    x2 = _td(x1, posT, pos1_flat, params['td0'], P0, NS1)

    x3 = _tb_block(x2, pos1_flat, pos1T, params['tb1'], NS1, 128)

    NS2 = 64
    sx2, sy2 = _fps(sx1, sy1, NS2)
    pos2 = jnp.stack([sx2, sy2], axis=-1)
    pos2_flat = pos2.reshape(B * NS2, 2)
    pos2T = jnp.stack([sx2, sy2], axis=1)
    x4 = _td(x3, pos1T, pos2_flat, params['td1'], NS1, NS2)

    x5 = _tb_block(x4, pos2_flat, pos2T, params['tb2'], NS2, 256)

    return _head(x5, params, NS2)


# speedup vs baseline: 12.5511x; 1.2818x over previous
import functools

import jax
import jax.numpy as jnp
from jax.experimental import pallas as pl
from jax.experimental.pallas import tpu as pltpu

B = 16
P0 = 1024
K = 16
BIG = 1e10
F32 = jnp.float32


def _dot(a, b):
    return jnp.dot(a, b, preferred_element_type=F32)


def _relu(a):
    return jnp.maximum(a, 0.0)


def _tb_kernel(x_ref, pos_ref, posT_ref,
               miW_ref, mib_ref,
               Win_ref, bin_ref, Wout_ref, bout_ref,
               Wlin_ref, Wsrc_ref, Wdst_ref,
               pW1_ref, pb1_ref, pW2_ref, pb2_ref,
               aW1_ref, ab1_ref, aW2_ref, ab2_ref,
               o_ref, *, P, C, has_mlp_in):
    x = x_ref[...]
    if has_mlp_in:
        miW = miW_ref[...]
        x = _relu(x[:, 0:1] * miW[0:1, :] + x[:, 1:2] * miW[1:2, :]
                  + x[:, 2:3] * miW[2:3, :] + mib_ref[...])
    xin = _relu(_dot(x, Win_ref[...]) + bin_ref[...])
    a_dst = _dot(xin, Wdst_ref[...])

    px = pos_ref[:, 0:1]
    py = pos_ref[:, 1:2]
    dx = px - posT_ref[0, 0:1, :]
    dy = py - posT_ref[0, 1:2, :]
    ii = jax.lax.broadcasted_iota(jnp.int32, (P, P), 0)
    jj = jax.lax.broadcasted_iota(jnp.int32, (P, P), 1)
    d = dx * dx + dy * dy + jnp.where(ii == jj, BIG, 0.0)
    jjf = jj.astype(F32)

    G = jnp.concatenate([pos_ref[...], xin], axis=1)

    pW1 = pW1_ref[...]
    pb1 = pb1_ref[...]
    pW2 = pW2_ref[...]
    pb2 = pb2_ref[...]
    aW1 = aW1_ref[...]
    ab1 = ab1_ref[...]
    aW2 = aW2_ref[...]
    ab2 = ab2_ref[...]
    Wsrc = Wsrc_ref[...]
    Wlin = Wlin_ref[...]

    Wsl = jnp.concatenate([Wsrc, Wlin], axis=1)

    def slot(g):
        pdx = px - g[:, 0:1]
        pdy = py - g[:, 1:2]
        gxin = g[:, 2:]
        h1 = _relu(pdx * pW1[0:1, :] + pdy * pW1[1:2, :] + pb1)
        delta = _relu(_dot(h1, pW2) + pb2)
        sl = _dot(gxin, Wsl)
        asrc = sl[:, :C]
        xl = sl[:, C:]
        ah = _relu(_dot(a_dst - asrc + delta, aW1) + ab1)
        alpha = _relu(_dot(ah, aW2) + ab2)
        return alpha, xl + delta

    alpha0, msg0 = slot(G)
    amax0 = alpha0
    den0 = jnp.ones((P, C), F32)
    num0 = msg0

    def body(t, carry):
        d, amax, den, num = carry
        m = jnp.min(d, axis=1, keepdims=True)
        cand = jnp.where(d == m, jjf, float(P))
        jsel = jnp.min(cand, axis=1, keepdims=True)
        oh = (jjf == jsel).astype(F32)
        g = _dot(oh, G)
        d = jnp.where(oh != 0.0, BIG, d)
        alpha, msg = slot(g)
        namax = jnp.maximum(amax, alpha)
        scale = jnp.exp(amax - namax)
        e = jnp.exp(alpha - namax)
        den = den * scale + e
        num = jnp.maximum(num * scale, e * msg)
        return d, namax, den, num

    _, _, den, num = jax.lax.fori_loop(0, K, body, (d, amax0, den0, num0),
                                       unroll=4)
    out = num / (den + 1e-16)
    o_ref[...] = _relu(_dot(out, Wout_ref[...]) + bout_ref[...])


def _tb_block(x, pos_flat, posT, p, P, C, mlp_in=None):
    Cin = x.shape[1]
    if mlp_in is None:
        miW = jnp.zeros((4, C), F32)
        mib = jnp.zeros((1, C), F32)
        has_mlp_in = False
    else:
        miW, mib = mlp_in
        mib = mib.reshape(1, -1)
        has_mlp_in = True

    def b2(v):
        return v.reshape(1, -1)

    ws = [miW, mib,
          p['Win'], b2(p['bin']), p['Wout'], b2(p['bout']),
          p['Wlin'], p['Wsrc'], p['Wdst'],
          p['pW1'], b2(p['pb1']), p['pW2'], b2(p['pb2']),
          p['aW1'], b2(p['ab1']), p['aW2'], b2(p['ab2'])]
    w_specs = [pl.BlockSpec(w.shape, lambda b: (0, 0)) for w in ws]
    return pl.pallas_call(
        functools.partial(_tb_kernel, P=P, C=C, has_mlp_in=has_mlp_in),
        grid=(B,),
        in_specs=[pl.BlockSpec((P, Cin), lambda b: (b, 0)),
                  pl.BlockSpec((P, 2), lambda b: (b, 0)),
                  pl.BlockSpec((1, 2, P), lambda b: (b, 0, 0))] + w_specs,
        out_specs=pl.BlockSpec((P, C), lambda b: (b, 0)),
        out_shape=jax.ShapeDtypeStruct((B * P, C), F32),
    )(x, pos_flat, posT, *ws)


def _fps_kernel(pbx_ref, pby_ref, ox_ref, oy_ref, *, Pc, NS):
    pbx = pbx_ref[...]
    pby = pby_ref[...]
    jjf = jax.lax.broadcasted_iota(jnp.int32, (B, Pc), 1).astype(F32)
    sel = jax.lax.broadcasted_iota(jnp.int32, (B, NS), 1).astype(F32)

    lpx0 = pbx[:, 0:1]
    lpy0 = pby[:, 0:1]
    posx0 = jnp.where(sel == 0.0, lpx0, 0.0)
    posy0 = jnp.where(sel == 0.0, lpy0, 0.0)

    def body(i, carry):
        dists, lpx, lpy, posx, posy = carry
        ddx = pbx - lpx
        ddy = pby - lpy
        dcur = ddx * ddx + ddy * ddy
        dists = jnp.minimum(dists, dcur)
        mx = jnp.max(dists, axis=1, keepdims=True)
        cand = jnp.where(dists == mx, jjf, float(Pc))
        jselv = jnp.min(cand, axis=1, keepdims=True)
        oh = (jjf == jselv).astype(F32)
        lpx = jnp.sum(pbx * oh, axis=1, keepdims=True)
        lpy = jnp.sum(pby * oh, axis=1, keepdims=True)
        slot = (sel == i.astype(F32)).astype(F32)
        posx = posx + slot * lpx
        posy = posy + slot * lpy
        return dists, lpx, lpy, posx, posy

    init = (jnp.full((B, Pc), jnp.inf, F32), lpx0, lpy0, posx0, posy0)
    _, _, _, posx, posy = jax.lax.fori_loop(1, NS, body, init)
    ox_ref[...] = posx
    oy_ref[...] = posy


def _fps(pbx, pby, NS):
    Pc = pbx.shape[1]
    return pl.pallas_call(
        functools.partial(_fps_kernel, Pc=Pc, NS=NS),
        in_specs=[pl.BlockSpec((B, Pc), lambda: (0, 0)),
                  pl.BlockSpec((B, Pc), lambda: (0, 0))],
        out_specs=[pl.BlockSpec((B, NS), lambda: (0, 0)),
                   pl.BlockSpec((B, NS), lambda: (0, 0))],
        out_shape=[jax.ShapeDtypeStruct((B, NS), F32),
                   jax.ShapeDtypeStruct((B, NS), F32)],
    )(pbx, pby)


def _td_kernel(x_ref, posT_ref, q_ref, W_ref, b_ref, o_ref, *, P, NS):
    h = _dot(x_ref[...], W_ref[...]) + b_ref[...]
    m = jnp.mean(h, axis=0, keepdims=True)
    v = jnp.mean((h - m) ** 2, axis=0, keepdims=True)
    hn = _relu((h - m) / jnp.sqrt(v + 1e-5))

    qx = q_ref[:, 0:1]
    qy = q_ref[:, 1:2]
    ddx = qx - posT_ref[0, 0:1, :]
    ddy = qy - posT_ref[0, 1:2, :]
    d = ddx * ddx + ddy * ddy
    jjf = jax.lax.broadcasted_iota(jnp.int32, (NS, P), 1).astype(F32)

    Cout = hn.shape[1]

    def body(t, carry):
        d, pooled = carry
        mn = jnp.min(d, axis=1, keepdims=True)
        cand = jnp.where(d == mn, jjf, float(P))
        jsel = jnp.min(cand, axis=1, keepdims=True)
        oh = (jjf == jsel).astype(F32)
        g = _dot(oh, hn)
        pooled = jnp.maximum(pooled, g)
        d = jnp.where(oh != 0.0, BIG, d)
        return d, pooled

    init = (d, jnp.full((NS, Cout), -jnp.inf, F32))
    _, pooled = jax.lax.fori_loop(0, K + 1, body, init, unroll=4)
    o_ref[...] = pooled


def _td(x, posT, q_flat, Wb, P, NS):
    Cin = x.shape[1]
    Cout = Wb['W'].shape[1]
    b2 = Wb['b'].reshape(1, -1)
    return pl.pallas_call(
        functools.partial(_td_kernel, P=P, NS=NS),
        grid=(B,),
        in_specs=[pl.BlockSpec((P, Cin), lambda b: (b, 0)),
                  pl.BlockSpec((1, 2, P), lambda b: (b, 0, 0)),
                  pl.BlockSpec((NS, 2), lambda b: (b, 0)),
                  pl.BlockSpec(Wb['W'].shape, lambda b: (0, 0)),
                  pl.BlockSpec(b2.shape, lambda b: (0, 0))],
        out_specs=pl.BlockSpec((NS, Cout), lambda b: (b, 0)),
        out_shape=jax.ShapeDtypeStruct((B * NS, Cout), F32),
    )(x, posT, q_flat, Wb['W'], b2)


def _head_kernel(x_ref, W1_ref, b1_ref, W2_ref, b2_ref, o_ref, *, NP):
    x = x_ref[...]
    row = jax.lax.broadcasted_iota(jnp.int32, (B, B * NP), 0)
    col = jax.lax.broadcasted_iota(jnp.int32, (B, B * NP), 1)
    Msel = jnp.where(col // NP == row, 1.0, 0.0)
    xg = _dot(Msel, x) / float(NP)
    h = _relu(_dot(xg, W1_ref[...]) + b1_ref[...])
    o_ref[...] = _dot(h, W2_ref[...]) + b2_ref[...]


def _head(x, params, NP):
    C = x.shape[1]
    W1 = params['mo_W1']
    b1 = params['mo_b1'].reshape(1, -1)
    W2 = params['mo_W2']
    b2 = params['mo_b2'].reshape(1, -1)
    OUT = W2.shape[1]
    return pl.pallas_call(
        functools.partial(_head_kernel, NP=NP),
        in_specs=[pl.BlockSpec((B * NP, C), lambda: (0, 0)),
                  pl.BlockSpec(W1.shape, lambda: (0, 0)),
                  pl.BlockSpec(b1.shape, lambda: (0, 0)),
                  pl.BlockSpec(W2.shape, lambda: (0, 0)),
                  pl.BlockSpec(b2.shape, lambda: (0, 0))],
        out_specs=pl.BlockSpec((B, OUT), lambda: (0, 0)),
        out_shape=jax.ShapeDtypeStruct((B, OUT), F32),
    )(x, W1, b1, W2, b2)


def kernel(x, pos, batch, params):
    pb = pos.reshape(B, P0, 2)
    posT = pb.transpose(0, 2, 1)
    pbx = pb[:, :, 0]
    pby = pb[:, :, 1]

    x1 = _tb_block(x, pos, posT, params['tb0'], P0, 64,
                   mlp_in=(params['mi_W'], params['mi_b']))

    NS1 = 256
    sx1, sy1 = _fps(pbx, pby, NS1)
    pos1 = jnp.stack([sx1, sy1], axis=-1)
    pos1_flat = pos1.reshape(B * NS1, 2)
    pos1T = jnp.stack([sx1, sy1], axis=1)
    x2 = _td(x1, posT, pos1_flat, params['td0'], P0, NS1)

    x3 = _tb_block(x2, pos1_flat, pos1T, params['tb1'], NS1, 128)

    NS2 = 64
    sx2, sy2 = _fps(sx1, sy1, NS2)
    pos2 = jnp.stack([sx2, sy2], axis=-1)
    pos2_flat = pos2.reshape(B * NS2, 2)
    pos2T = jnp.stack([sx2, sy2], axis=1)
    x4 = _td(x3, pos1T, pos2_flat, params['td1'], NS1, NS2)

    x5 = _tb_block(x4, pos2_flat, pos2T, params['tb2'], NS2, 256)

    return _head(x5, params, NS2)


# speedup vs baseline: 13.5871x; 1.0825x over previous
import functools

import jax
import jax.numpy as jnp
from jax.experimental import pallas as pl
from jax.experimental.pallas import tpu as pltpu

B = 16
P0 = 1024
K = 16
BIG = 1e10
F32 = jnp.float32


def _dot(a, b):
    return jnp.dot(a, b, preferred_element_type=F32)


def _relu(a):
    return jnp.maximum(a, 0.0)


def _tb_kernel(x_ref, pos_ref, posT_ref,
               miW_ref, mib_ref,
               Win_ref, bin_ref, Wout_ref, bout_ref,
               Wlin_ref, Wsrc_ref, Wdst_ref,
               pW1_ref, pb1_ref, pW2_ref, pb2_ref,
               aW1_ref, ab1_ref, aW2_ref, ab2_ref,
               o_ref, *, P, C, has_mlp_in):
    x = x_ref[...]
    if has_mlp_in:
        x = _relu(_dot(x, miW_ref[...]) + mib_ref[...])
    xin = _relu(_dot(x, Win_ref[...]) + bin_ref[...])
    a_dst = _dot(xin, Wdst_ref[...])

    px = pos_ref[:, 0:1]
    py = pos_ref[:, 1:2]
    dx = px - posT_ref[0, 0:1, :]
    dy = py - posT_ref[0, 1:2, :]
    ii = jax.lax.broadcasted_iota(jnp.int32, (P, P), 0)
    jj = jax.lax.broadcasted_iota(jnp.int32, (P, P), 1)
    d = dx * dx + dy * dy + jnp.where(ii == jj, BIG, 0.0)
    jjf = jj.astype(F32)

    U = _dot(pos_ref[...], pW1_ref[...])
    G = jnp.concatenate([xin, U], axis=1)

    pb1 = pb1_ref[...]
    pW2 = pW2_ref[...]
    pb2 = pb2_ref[...]
    aW1 = aW1_ref[...]
    ab1 = ab1_ref[...]
    aW2 = aW2_ref[...]
    ab2 = ab2_ref[...]
    Wsrc = Wsrc_ref[...]
    Wlin = Wlin_ref[...]

    Wsl = jnp.concatenate([Wsrc, Wlin], axis=1)

    def slot(g):
        gxin = g[:, :C]
        gU = g[:, C:]
        h1 = _relu(U - gU + pb1)
        delta = _relu(_dot(h1, pW2) + pb2)
        sl = _dot(gxin, Wsl)
        asrc = sl[:, :C]
        xl = sl[:, C:]
        ah = _relu(_dot(a_dst - asrc + delta, aW1) + ab1)
        alpha = _relu(_dot(ah, aW2) + ab2)
        return alpha, xl + delta

    alpha0, msg0 = slot(G)
    amax0 = alpha0
    den0 = jnp.ones((P, C), F32)
    num0 = msg0

    def body(t, carry):
        d, amax, den, num = carry
        m = jnp.min(d, axis=1, keepdims=True)
        cand = jnp.where(d == m, jjf, float(P))
        jsel = jnp.min(cand, axis=1, keepdims=True)
        oh = (jjf == jsel).astype(F32)
        g = _dot(oh, G)
        d = jnp.where(oh != 0.0, BIG, d)
        alpha, msg = slot(g)
        namax = jnp.maximum(amax, alpha)
        scale = jnp.exp(amax - namax)
        e = jnp.exp(alpha - namax)
        den = den * scale + e
        num = jnp.maximum(num * scale, e * msg)
        return d, namax, den, num

    _, _, den, num = jax.lax.fori_loop(0, K, body, (d, amax0, den0, num0),
                                       unroll=4)
    out = num / (den + 1e-16)
    o_ref[...] = _relu(_dot(out, Wout_ref[...]) + bout_ref[...])


def _tb_block(x, pos_flat, posT, p, P, C, mlp_in=None):
    Cin = x.shape[1]
    if mlp_in is None:
        miW = jnp.zeros((4, C), F32)
        mib = jnp.zeros((1, C), F32)
        has_mlp_in = False
    else:
        miW, mib = mlp_in
        mib = mib.reshape(1, -1)
        has_mlp_in = True

    def b2(v):
        return v.reshape(1, -1)

    ws = [miW, mib,
          p['Win'], b2(p['bin']), p['Wout'], b2(p['bout']),
          p['Wlin'], p['Wsrc'], p['Wdst'],
          p['pW1'], b2(p['pb1']), p['pW2'], b2(p['pb2']),
          p['aW1'], b2(p['ab1']), p['aW2'], b2(p['ab2'])]
    w_specs = [pl.BlockSpec(w.shape, lambda b: (0, 0)) for w in ws]
    return pl.pallas_call(
        functools.partial(_tb_kernel, P=P, C=C, has_mlp_in=has_mlp_in),
        grid=(B,),
        in_specs=[pl.BlockSpec((P, Cin), lambda b: (b, 0)),
                  pl.BlockSpec((P, 2), lambda b: (b, 0)),
                  pl.BlockSpec((1, 2, P), lambda b: (b, 0, 0))] + w_specs,
        out_specs=pl.BlockSpec((P, C), lambda b: (b, 0)),
        out_shape=jax.ShapeDtypeStruct((B * P, C), F32),
    )(x, pos_flat, posT, *ws)


def _fps_kernel(pbx_ref, pby_ref, ox_ref, oy_ref, *, Pc, NS):
    pbx = pbx_ref[...]
    pby = pby_ref[...]
    jjf = jax.lax.broadcasted_iota(jnp.int32, (B, Pc), 1).astype(F32)
    sel = jax.lax.broadcasted_iota(jnp.int32, (B, NS), 1).astype(F32)

    lpx0 = pbx[:, 0:1]
    lpy0 = pby[:, 0:1]
    posx0 = jnp.where(sel == 0.0, lpx0, 0.0)
    posy0 = jnp.where(sel == 0.0, lpy0, 0.0)

    def body(i, carry):
        dists, lpx, lpy, posx, posy = carry
        ddx = pbx - lpx
        ddy = pby - lpy
        dcur = ddx * ddx + ddy * ddy
        dists = jnp.minimum(dists, dcur)
        mx = jnp.max(dists, axis=1, keepdims=True)
        cand = jnp.where(dists == mx, jjf, float(Pc))
        jselv = jnp.min(cand, axis=1, keepdims=True)
        oh = (jjf == jselv).astype(F32)
        lpx = jnp.sum(pbx * oh, axis=1, keepdims=True)
        lpy = jnp.sum(pby * oh, axis=1, keepdims=True)
        slot = (sel == i.astype(F32)).astype(F32)
        posx = posx + slot * lpx
        posy = posy + slot * lpy
        return dists, lpx, lpy, posx, posy

    init = (jnp.full((B, Pc), jnp.inf, F32), lpx0, lpy0, posx0, posy0)
    _, _, _, posx, posy = jax.lax.fori_loop(1, NS, body, init)
    ox_ref[...] = posx
    oy_ref[...] = posy


def _fps(pbx, pby, NS):
    Pc = pbx.shape[1]
    return pl.pallas_call(
        functools.partial(_fps_kernel, Pc=Pc, NS=NS),
        in_specs=[pl.BlockSpec((B, Pc), lambda: (0, 0)),
                  pl.BlockSpec((B, Pc), lambda: (0, 0))],
        out_specs=[pl.BlockSpec((B, NS), lambda: (0, 0)),
                   pl.BlockSpec((B, NS), lambda: (0, 0))],
        out_shape=[jax.ShapeDtypeStruct((B, NS), F32),
                   jax.ShapeDtypeStruct((B, NS), F32)],
    )(pbx, pby)


def _td_kernel(x_ref, posT_ref, q_ref, W_ref, b_ref, o_ref, *, P, NS):
    h = _dot(x_ref[...], W_ref[...]) + b_ref[...]
    m = jnp.mean(h, axis=0, keepdims=True)
    v = jnp.mean((h - m) ** 2, axis=0, keepdims=True)
    hn = _relu((h - m) / jnp.sqrt(v + 1e-5))

    qx = q_ref[:, 0:1]
    qy = q_ref[:, 1:2]
    ddx = qx - posT_ref[0, 0:1, :]
    ddy = qy - posT_ref[0, 1:2, :]
    d = ddx * ddx + ddy * ddy
    jjf = jax.lax.broadcasted_iota(jnp.int32, (NS, P), 1).astype(F32)

    Cout = hn.shape[1]

    def body(t, carry):
        d, pooled = carry
        mn = jnp.min(d, axis=1, keepdims=True)
        cand = jnp.where(d == mn, jjf, float(P))
        jsel = jnp.min(cand, axis=1, keepdims=True)
        oh = (jjf == jsel).astype(F32)
        g = _dot(oh, hn)
        pooled = jnp.maximum(pooled, g)
        d = jnp.where(oh != 0.0, BIG, d)
        return d, pooled

    init = (d, jnp.full((NS, Cout), -jnp.inf, F32))
    _, pooled = jax.lax.fori_loop(0, K + 1, body, init, unroll=4)
    o_ref[...] = pooled


def _td(x, posT, q_flat, Wb, P, NS):
    Cin = x.shape[1]
    Cout = Wb['W'].shape[1]
    b2 = Wb['b'].reshape(1, -1)
    return pl.pallas_call(
        functools.partial(_td_kernel, P=P, NS=NS),
        grid=(B,),
        in_specs=[pl.BlockSpec((P, Cin), lambda b: (b, 0)),
                  pl.BlockSpec((1, 2, P), lambda b: (b, 0, 0)),
                  pl.BlockSpec((NS, 2), lambda b: (b, 0)),
                  pl.BlockSpec(Wb['W'].shape, lambda b: (0, 0)),
                  pl.BlockSpec(b2.shape, lambda b: (0, 0))],
        out_specs=pl.BlockSpec((NS, Cout), lambda b: (b, 0)),
        out_shape=jax.ShapeDtypeStruct((B * NS, Cout), F32),
    )(x, posT, q_flat, Wb['W'], b2)


def _head_kernel(x_ref, W1_ref, b1_ref, W2_ref, b2_ref, o_ref, *, NP):
    x = x_ref[...]
    row = jax.lax.broadcasted_iota(jnp.int32, (B, B * NP), 0)
    col = jax.lax.broadcasted_iota(jnp.int32, (B, B * NP), 1)
    Msel = jnp.where(col // NP == row, 1.0, 0.0)
    xg = _dot(Msel, x) / float(NP)
    h = _relu(_dot(xg, W1_ref[...]) + b1_ref[...])
    o_ref[...] = _dot(h, W2_ref[...]) + b2_ref[...]


def _head(x, params, NP):
    C = x.shape[1]
    W1 = params['mo_W1']
    b1 = params['mo_b1'].reshape(1, -1)
    W2 = params['mo_W2']
    b2 = params['mo_b2'].reshape(1, -1)
    OUT = W2.shape[1]
    return pl.pallas_call(
        functools.partial(_head_kernel, NP=NP),
        in_specs=[pl.BlockSpec((B * NP, C), lambda: (0, 0)),
                  pl.BlockSpec(W1.shape, lambda: (0, 0)),
                  pl.BlockSpec(b1.shape, lambda: (0, 0)),
                  pl.BlockSpec(W2.shape, lambda: (0, 0)),
                  pl.BlockSpec(b2.shape, lambda: (0, 0))],
        out_specs=pl.BlockSpec((B, OUT), lambda: (0, 0)),
        out_shape=jax.ShapeDtypeStruct((B, OUT), F32),
    )(x, W1, b1, W2, b2)


def kernel(x, pos, batch, params):
    pb = pos.reshape(B, P0, 2)
    posT = pb.transpose(0, 2, 1)
    pbx = pb[:, :, 0]
    pby = pb[:, :, 1]

    x1 = _tb_block(x, pos, posT, params['tb0'], P0, 64,
                   mlp_in=(params['mi_W'], params['mi_b']))

    NS1 = 256
    sx1, sy1 = _fps(pbx, pby, NS1)
    pos1 = jnp.stack([sx1, sy1], axis=-1)
    pos1_flat = pos1.reshape(B * NS1, 2)
    pos1T = jnp.stack([sx1, sy1], axis=1)
    x2 = _td(x1, posT, pos1_flat, params['td0'], P0, NS1)

    x3 = _tb_block(x2, pos1_flat, pos1T, params['tb1'], NS1, 128)

    NS2 = 64
    sx2, sy2 = _fps(sx1, sy1, NS2)
    pos2 = jnp.stack([sx2, sy2], axis=-1)
    pos2_flat = pos2.reshape(B * NS2, 2)
    pos2T = jnp.stack([sx2, sy2], axis=1)
    x4 = _td(x3, pos1T, pos2_flat, params['td1'], NS1, NS2)

    x5 = _tb_block(x4, pos2_flat, pos2T, params['tb2'], NS2, 256)

    return _head(x5, params, NS2)


# speedup vs baseline: 13.8166x; 1.0169x over previous
import functools

import jax
import jax.numpy as jnp
from jax.experimental import pallas as pl
from jax.experimental.pallas import tpu as pltpu

B = 16
P0 = 1024
K = 16
BIG = 1e10
F32 = jnp.float32


def _dot(a, b):
    return jnp.dot(a, b, preferred_element_type=F32)


def _relu(a):
    return jnp.maximum(a, 0.0)


def _tb_kernel(x_ref, pos_ref, posT_ref,
               miW_ref, mib_ref,
               Win_ref, bin_ref, Wout_ref, bout_ref,
               Wlin_ref, Wsrc_ref, Wdst_ref,
               pW1_ref, pb1_ref, pW2_ref, pb2_ref,
               aW1_ref, ab1_ref, aW2_ref, ab2_ref,
               o_ref, *, P, C, has_mlp_in):
    x = x_ref[...]
    if has_mlp_in:
        x = _relu(_dot(x, miW_ref[...]) + mib_ref[...])
    xin = _relu(_dot(x, Win_ref[...]) + bin_ref[...])
    a_dst = _dot(xin, Wdst_ref[...])

    px = pos_ref[:, 0:1]
    py = pos_ref[:, 1:2]
    dx = px - posT_ref[0, 0:1, :]
    dy = py - posT_ref[0, 1:2, :]
    ii = jax.lax.broadcasted_iota(jnp.int32, (P, P), 0)
    jj = jax.lax.broadcasted_iota(jnp.int32, (P, P), 1)
    d = dx * dx + dy * dy + jnp.where(ii == jj, BIG, 0.0)
    jjf = jj.astype(F32)

    U = _dot(pos_ref[...], pW1_ref[...])
    G = jnp.concatenate([xin, U], axis=1)

    pb1 = pb1_ref[...]
    pW2 = pW2_ref[...]
    pb2 = pb2_ref[...]
    aW1 = aW1_ref[...]
    ab1 = ab1_ref[...]
    aW2 = aW2_ref[...]
    ab2 = ab2_ref[...]
    Wsrc = Wsrc_ref[...]
    Wlin = Wlin_ref[...]

    Wsl = jnp.concatenate([Wsrc, Wlin], axis=1)

    def slot(g):
        gxin = g[:, :C]
        gU = g[:, C:]
        h1 = _relu(U - gU + pb1)
        delta = _relu(_dot(h1, pW2) + pb2)
        sl = _dot(gxin, Wsl)
        asrc = sl[:, :C]
        xl = sl[:, C:]
        ah = _relu(_dot(a_dst - asrc + delta, aW1) + ab1)
        alpha = _relu(_dot(ah, aW2) + ab2)
        return alpha, xl + delta

    alpha0, msg0 = slot(G)
    amax0 = alpha0
    den0 = jnp.ones((P, C), F32)
    num0 = msg0

    def body(t, carry):
        d, amax, den, num = carry
        m = jnp.min(d, axis=1, keepdims=True)
        cand = jnp.where(d == m, jjf, float(P))
        jsel = jnp.min(cand, axis=1, keepdims=True)
        ohb = jjf == jsel
        g = _dot(ohb.astype(F32), G)
        d = jnp.where(ohb, BIG, d)
        alpha, msg = slot(g)
        namax = jnp.maximum(amax, alpha)
        scale = jnp.exp(amax - namax)
        e = jnp.exp(alpha - namax)
        den = den * scale + e
        num = jnp.maximum(num * scale, e * msg)
        return d, namax, den, num

    _, _, den, num = jax.lax.fori_loop(0, K, body, (d, amax0, den0, num0),
                                       unroll=4)
    out = num / (den + 1e-16)
    o_ref[...] = _relu(_dot(out, Wout_ref[...]) + bout_ref[...])


def _tb_block(x, pos_flat, posT, p, P, C, mlp_in=None):
    Cin = x.shape[1]
    if mlp_in is None:
        miW = jnp.zeros((4, C), F32)
        mib = jnp.zeros((1, C), F32)
        has_mlp_in = False
    else:
        miW, mib = mlp_in
        mib = mib.reshape(1, -1)
        has_mlp_in = True

    def b2(v):
        return v.reshape(1, -1)

    ws = [miW, mib,
          p['Win'], b2(p['bin']), p['Wout'], b2(p['bout']),
          p['Wlin'], p['Wsrc'], p['Wdst'],
          p['pW1'], b2(p['pb1']), p['pW2'], b2(p['pb2']),
          p['aW1'], b2(p['ab1']), p['aW2'], b2(p['ab2'])]
    w_specs = [pl.BlockSpec(w.shape, lambda b: (0, 0)) for w in ws]
    return pl.pallas_call(
        functools.partial(_tb_kernel, P=P, C=C, has_mlp_in=has_mlp_in),
        grid=(B,),
        in_specs=[pl.BlockSpec((P, Cin), lambda b: (b, 0)),
                  pl.BlockSpec((P, 2), lambda b: (b, 0)),
                  pl.BlockSpec((1, 2, P), lambda b: (b, 0, 0))] + w_specs,
        out_specs=pl.BlockSpec((P, C), lambda b: (b, 0)),
        out_shape=jax.ShapeDtypeStruct((B * P, C), F32),
    )(x, pos_flat, posT, *ws)


def _fps_kernel(pbx_ref, pby_ref, ox_ref, oy_ref, *, Pc, NS):
    pbx = pbx_ref[...]
    pby = pby_ref[...]
    jjf = jax.lax.broadcasted_iota(jnp.int32, (B, Pc), 1).astype(F32)
    sel = jax.lax.broadcasted_iota(jnp.int32, (B, NS), 1).astype(F32)

    lpx0 = pbx[:, 0:1]
    lpy0 = pby[:, 0:1]
    posx0 = jnp.where(sel == 0.0, lpx0, 0.0)
    posy0 = jnp.where(sel == 0.0, lpy0, 0.0)

    def body(i, carry):
        dists, lpx, lpy, posx, posy = carry
        ddx = pbx - lpx
        ddy = pby - lpy
        dcur = ddx * ddx + ddy * ddy
        dists = jnp.minimum(dists, dcur)
        mx = jnp.max(dists, axis=1, keepdims=True)
        cand = jnp.where(dists == mx, jjf, float(Pc))
        jselv = jnp.min(cand, axis=1, keepdims=True)
        oh = (jjf == jselv).astype(F32)
        lpx = jnp.sum(pbx * oh, axis=1, keepdims=True)
        lpy = jnp.sum(pby * oh, axis=1, keepdims=True)
        slot = (sel == i.astype(F32)).astype(F32)
        posx = posx + slot * lpx
        posy = posy + slot * lpy
        return dists, lpx, lpy, posx, posy

    init = (jnp.full((B, Pc), jnp.inf, F32), lpx0, lpy0, posx0, posy0)
    _, _, _, posx, posy = jax.lax.fori_loop(1, NS, body, init)
    ox_ref[...] = posx
    oy_ref[...] = posy


def _fps(pbx, pby, NS):
    Pc = pbx.shape[1]
    return pl.pallas_call(
        functools.partial(_fps_kernel, Pc=Pc, NS=NS),
        in_specs=[pl.BlockSpec((B, Pc), lambda: (0, 0)),
                  pl.BlockSpec((B, Pc), lambda: (0, 0))],
        out_specs=[pl.BlockSpec((B, NS), lambda: (0, 0)),
                   pl.BlockSpec((B, NS), lambda: (0, 0))],
        out_shape=[jax.ShapeDtypeStruct((B, NS), F32),
                   jax.ShapeDtypeStruct((B, NS), F32)],
    )(pbx, pby)


def _td_kernel(x_ref, posT_ref, q_ref, W_ref, b_ref, o_ref, *, P, NS):
    h = _dot(x_ref[...], W_ref[...]) + b_ref[...]
    m = jnp.mean(h, axis=0, keepdims=True)
    v = jnp.mean((h - m) ** 2, axis=0, keepdims=True)
    hn = _relu((h - m) / jnp.sqrt(v + 1e-5))

    qx = q_ref[:, 0:1]
    qy = q_ref[:, 1:2]
    ddx = qx - posT_ref[0, 0:1, :]
    ddy = qy - posT_ref[0, 1:2, :]
    d = ddx * ddx + ddy * ddy
    jjf = jax.lax.broadcasted_iota(jnp.int32, (NS, P), 1).astype(F32)

    Cout = hn.shape[1]

    def body(t, carry):
        d, pooled = carry
        mn = jnp.min(d, axis=1, keepdims=True)
        cand = jnp.where(d == mn, jjf, float(P))
        jsel = jnp.min(cand, axis=1, keepdims=True)
        ohb = jjf == jsel
        g = _dot(ohb.astype(F32), hn)
        pooled = jnp.maximum(pooled, g)
        d = jnp.where(ohb, BIG, d)
        return d, pooled

    init = (d, jnp.full((NS, Cout), -jnp.inf, F32))
    _, pooled = jax.lax.fori_loop(0, K + 1, body, init, unroll=4)
    o_ref[...] = pooled


def _td(x, posT, q_flat, Wb, P, NS):
    Cin = x.shape[1]
    Cout = Wb['W'].shape[1]
    b2 = Wb['b'].reshape(1, -1)
    return pl.pallas_call(
        functools.partial(_td_kernel, P=P, NS=NS),
        grid=(B,),
        in_specs=[pl.BlockSpec((P, Cin), lambda b: (b, 0)),
                  pl.BlockSpec((1, 2, P), lambda b: (b, 0, 0)),
                  pl.BlockSpec((NS, 2), lambda b: (b, 0)),
                  pl.BlockSpec(Wb['W'].shape, lambda b: (0, 0)),
                  pl.BlockSpec(b2.shape, lambda b: (0, 0))],
        out_specs=pl.BlockSpec((NS, Cout), lambda b: (b, 0)),
        out_shape=jax.ShapeDtypeStruct((B * NS, Cout), F32),
    )(x, posT, q_flat, Wb['W'], b2)


def _head_kernel(x_ref, W1_ref, b1_ref, W2_ref, b2_ref, o_ref, *, NP):
    x = x_ref[...]
    row = jax.lax.broadcasted_iota(jnp.int32, (B, B * NP), 0)
    col = jax.lax.broadcasted_iota(jnp.int32, (B, B * NP), 1)
    Msel = jnp.where(col // NP == row, 1.0, 0.0)
    xg = _dot(Msel, x) / float(NP)
    h = _relu(_dot(xg, W1_ref[...]) + b1_ref[...])
    o_ref[...] = _dot(h, W2_ref[...]) + b2_ref[...]


def _head(x, params, NP):
    C = x.shape[1]
    W1 = params['mo_W1']
    b1 = params['mo_b1'].reshape(1, -1)
    W2 = params['mo_W2']
    b2 = params['mo_b2'].reshape(1, -1)
    OUT = W2.shape[1]
    return pl.pallas_call(
        functools.partial(_head_kernel, NP=NP),
        in_specs=[pl.BlockSpec((B * NP, C), lambda: (0, 0)),
                  pl.BlockSpec(W1.shape, lambda: (0, 0)),
                  pl.BlockSpec(b1.shape, lambda: (0, 0)),
                  pl.BlockSpec(W2.shape, lambda: (0, 0)),
                  pl.BlockSpec(b2.shape, lambda: (0, 0))],
        out_specs=pl.BlockSpec((B, OUT), lambda: (0, 0)),
        out_shape=jax.ShapeDtypeStruct((B, OUT), F32),
    )(x, W1, b1, W2, b2)


def kernel(x, pos, batch, params):
    pb = pos.reshape(B, P0, 2)
    posT = pb.transpose(0, 2, 1)
    pbx = pb[:, :, 0]
    pby = pb[:, :, 1]

    x1 = _tb_block(x, pos, posT, params['tb0'], P0, 64,
                   mlp_in=(params['mi_W'], params['mi_b']))

    NS1 = 256
    sx1, sy1 = _fps(pbx, pby, NS1)
    pos1 = jnp.stack([sx1, sy1], axis=-1)
    pos1_flat = pos1.reshape(B * NS1, 2)
    pos1T = jnp.stack([sx1, sy1], axis=1)
    x2 = _td(x1, posT, pos1_flat, params['td0'], P0, NS1)

    x3 = _tb_block(x2, pos1_flat, pos1T, params['tb1'], NS1, 128)

    NS2 = 64
    sx2, sy2 = _fps(sx1, sy1, NS2)
    pos2 = jnp.stack([sx2, sy2], axis=-1)
    pos2_flat = pos2.reshape(B * NS2, 2)
    pos2T = jnp.stack([sx2, sy2], axis=1)
    x4 = _td(x3, pos1T, pos2_flat, params['td1'], NS1, NS2)

    x5 = _tb_block(x4, pos2_flat, pos2T, params['tb2'], NS2, 256)

    return _head(x5, params, NS2)
